# baseline (speedup 1.0000x reference)
import functools

import jax
import jax.numpy as jnp
from jax import lax
from jax.experimental import pallas as pl
from jax.experimental.pallas import tpu as pltpu
from jax.experimental.pallas import tpu_sc as plsc

_B, _T, _L, _V = 1024, 200, 32, 100000
_NST, _NKC, _NKR, _NO = 4, 2, 4, 3
_BT = _B * _T
_NC, _NSUB = 2, 16
_NW = _NC * _NSUB
_CH = 128
_SROWS = (_B * _NST) // _NW
_KROWS = (_NKC * _BT) // _NW
_KCHUNKS = _KROWS // _CH


def _sc_gather(sflat, kflat, sidx, kidx):
    mesh = plsc.VectorSubcoreMesh(
        core_axis_name="c", subcore_axis_name="s",
        num_cores=_NC, num_subcores=_NSUB)

    @functools.partial(
        pl.kernel,
        out_type=(
            jax.ShapeDtypeStruct((_B * _NST, _L), jnp.float32),
            jax.ShapeDtypeStruct((_NKC * _BT, _L), jnp.float32),
        ),
        mesh=mesh,
        scratch_types=(
            pltpu.VMEM((_CH,), jnp.int32),
            pltpu.VMEM((_CH,), jnp.int32),
            pltpu.VMEM((_CH, _L), jnp.float32),
            pltpu.SemaphoreType.DMA,
        ),
        compiler_params=pltpu.CompilerParams(use_tc_tiling_on_sc=False),
    )
    def gk(sflat_h, kflat_h, sidx_h, kidx_h, sout_h, cout_h,
           sidx_v, kidx_v, rows_v, sem):
        w = lax.axis_index("s") * _NC + lax.axis_index("c")
        sbase = w * _SROWS
        pltpu.sync_copy(sidx_h.at[pl.ds(sbase, _CH)], sidx_v)
        pltpu.async_copy(sflat_h.at[sidx_v], rows_v, sem).wait()
        pltpu.sync_copy(rows_v, sout_h.at[pl.ds(sbase, _CH)])

        kbase = w * _KROWS

        def body(j, carry):
            off = kbase + j * _CH
            pltpu.sync_copy(kidx_h.at[pl.ds(off, _CH)], kidx_v)
            pltpu.async_copy(kflat_h.at[kidx_v], rows_v, sem).wait()
            pltpu.sync_copy(rows_v, cout_h.at[pl.ds(off, _CH)])
            return carry

        lax.fori_loop(0, _KCHUNKS, body, 0)

    return gk(sflat, kflat, sidx, kidx)


def _tc_assemble(kr2, obs2, cat3, wk, bk, wo, bo):
    blk = 1024
    hi = jax.lax.Precision.HIGHEST

    def body(kr_ref, obs_ref, cat_ref, wk_ref, bk_ref, wo_ref, bo_ref,
             kout_ref, oout_ref):
        acc = jax.lax.dot(kr_ref[...], wk_ref[0:_NKR, :], precision=hi)
        acc = acc + jax.lax.dot(cat_ref[0], wk_ref[_NKR:_NKR + _L, :],
                                precision=hi)
        acc = acc + jax.lax.dot(cat_ref[1], wk_ref[_NKR + _L:, :],
                                precision=hi)
        kout_ref[...] = acc + bk_ref[...]
        oout_ref[...] = (jax.lax.dot(obs_ref[...], wo_ref[...], precision=hi)
                         + bo_ref[...])

    kd = _NKR + _NKC * _L
    return pl.pallas_call(
        body,
        grid=(_BT // blk,),
        in_specs=[
            pl.BlockSpec((blk, _NKR), lambda i: (i, 0)),
            pl.BlockSpec((blk, _NO), lambda i: (i, 0)),
            pl.BlockSpec((_NKC, blk, _L), lambda i: (0, i, 0)),
            pl.BlockSpec((kd, 6 * _L), lambda i: (0, 0)),
            pl.BlockSpec((1, 6 * _L), lambda i: (0, 0)),
            pl.BlockSpec((_NO, _NO * _L), lambda i: (0, 0)),
            pl.BlockSpec((1, _NO * _L), lambda i: (0, 0)),
        ],
        out_specs=[
            pl.BlockSpec((blk, 6 * _L), lambda i: (i, 0)),
            pl.BlockSpec((blk, _NO * _L), lambda i: (i, 0)),
        ],
        out_shape=[
            jax.ShapeDtypeStruct((_BT, 6 * _L), jnp.float32),
            jax.ShapeDtypeStruct((_BT, _NO * _L), jnp.float32),
        ],
        compiler_params=pltpu.CompilerParams(
            dimension_semantics=("parallel",)),
    )(kr2, obs2, cat3, wk, bk, wo, bo)


def _build_weights(known_dense_w, known_dense_b, observed_dense_w,
                   observed_dense_b):
    l = jnp.arange(_L)
    i4 = jnp.arange(_NKR)
    wk = jnp.zeros((_NKR + _NKC * _L, 6 * _L), jnp.float32)
    wk = wk.at[i4[:, None], 6 * l[None, :] + i4[:, None]].set(
        known_dense_w[:, 0, :])
    wk = wk.at[_NKR + l, 6 * l + 4].set(1.0)
    wk = wk.at[_NKR + _L + l, 6 * l + 5].set(1.0)
    bk = jnp.zeros((6 * _L,), jnp.float32)
    bk = bk.at[(6 * l[None, :] + i4[:, None]).reshape(-1)].set(
        known_dense_b.reshape(-1))

    i3 = jnp.arange(_NO)
    wo = jnp.zeros((_NO, _NO * _L), jnp.float32)
    wo = wo.at[i3[:, None], _NO * l[None, :] + i3[:, None]].set(
        observed_dense_w[:, 0, :])
    bo = jnp.zeros((_NO * _L,), jnp.float32)
    bo = bo.at[(_NO * l[None, :] + i3[:, None]).reshape(-1)].set(
        observed_dense_b.reshape(-1))
    return wk, bk.reshape(1, -1), wo, bo.reshape(1, -1)


def kernel(static, known_real, known_categorical, observed, static_tables,
           known_tables, known_dense_w, known_dense_b, observed_dense_w,
           observed_dense_b):
    sflat = static_tables.reshape(_NST * _V, _L)
    kflat = known_tables.reshape(_NKC * _V, _L)

    sidx = (static[:, 0, :].astype(jnp.int32)
            + jnp.arange(_NST, dtype=jnp.int32)[None, :] * _V).reshape(-1)
    kidx = (known_categorical.reshape(_BT, _NKC).astype(jnp.int32)
            + jnp.arange(_NKC, dtype=jnp.int32)[None, :] * _V).T.reshape(-1)

    static_rows, cat_rows = _sc_gather(sflat, kflat, sidx, kidx)

    wk, bk, wo, bo = _build_weights(known_dense_w, known_dense_b,
                                    observed_dense_w, observed_dense_b)
    known2, obs2 = _tc_assemble(
        known_real.reshape(_BT, _NKR), observed.reshape(_BT, _NO),
        cat_rows.reshape(_NKC, _BT, _L), wk, bk, wo, bo)

    return (static_rows.reshape(_B, _NST, _L),
            known2.reshape(_B, _T, _L, 6),
            obs2.reshape(_B, _T, _L, _NO))


# speedup vs baseline: 2.8178x; 2.8178x over previous
import functools

import jax
import jax.numpy as jnp
from jax import lax
from jax.experimental import pallas as pl
from jax.experimental.pallas import tpu as pltpu
from jax.experimental.pallas import tpu_sc as plsc

_B, _T, _L, _V = 1024, 200, 32, 100000
_NST, _NKC, _NKR, _NO = 4, 2, 4, 3
_BT = _B * _T
_NC, _NSUB = 2, 16
_NW = _NC * _NSUB
_CH = 128
_SROWS = (_B * _NST) // _NW
_KROWS = (_NKC * _BT) // _NW
_KCHUNKS = _KROWS // _CH


def _sc_gather(sflat, kflat, sidx, kidx):
    mesh = plsc.VectorSubcoreMesh(
        core_axis_name="c", subcore_axis_name="s",
        num_cores=_NC, num_subcores=_NSUB)

    @functools.partial(
        pl.kernel,
        out_type=(
            jax.ShapeDtypeStruct((_B * _NST, _L), jnp.float32),
            jax.ShapeDtypeStruct((_NKC * _BT, _L), jnp.float32),
        ),
        mesh=mesh,
        scratch_types=(
            pltpu.VMEM((_CH,), jnp.int32),
            pltpu.VMEM((_CH,), jnp.int32),
            pltpu.VMEM((_CH, _L), jnp.float32),
            pltpu.SemaphoreType.DMA,
        ),
        compiler_params=pltpu.CompilerParams(use_tc_tiling_on_sc=False),
    )
    def gk(sflat_h, kflat_h, sidx_h, kidx_h, sout_h, cout_h,
           sidx_v, kidx_v, rows_v, sem):
        w = lax.axis_index("s") * _NC + lax.axis_index("c")
        sbase = w * _SROWS
        pltpu.sync_copy(sidx_h.at[pl.ds(sbase, _CH)], sidx_v)
        pltpu.async_copy(sflat_h.at[sidx_v], rows_v, sem).wait()
        pltpu.sync_copy(rows_v, sout_h.at[pl.ds(sbase, _CH)])

        kbase = w * _KROWS

        def body(j, carry):
            off = kbase + j * _CH
            pltpu.sync_copy(kidx_h.at[pl.ds(off, _CH)], kidx_v)
            pltpu.async_copy(kflat_h.at[kidx_v], rows_v, sem).wait()
            pltpu.sync_copy(rows_v, cout_h.at[pl.ds(off, _CH)])
            return carry

        lax.fori_loop(0, _KCHUNKS, body, 0)

    return gk(sflat, kflat, sidx, kidx)


def _tc_assemble(krT, obsT, cat4, wkT, bkT, woT, boT):
    def body(kr_ref, obs_ref, cat_ref, wk_ref, bk_ref, wo_ref, bo_ref,
             kout_ref, oout_ref):
        for j in range(_NKR):
            kout_ref[0, j] = wk_ref[j] * kr_ref[0, j][None, :] + bk_ref[j]
        for f in range(_NKC):
            kout_ref[0, _NKR + f] = jnp.transpose(cat_ref[0, f], (1, 0))
        for i in range(_NO):
            oout_ref[0, i] = wo_ref[i] * obs_ref[0, i][None, :] + bo_ref[i]

    return pl.pallas_call(
        body,
        grid=(_T,),
        in_specs=[
            pl.BlockSpec((1, _NKR, _B), lambda t: (t, 0, 0)),
            pl.BlockSpec((1, _NO, _B), lambda t: (t, 0, 0)),
            pl.BlockSpec((1, _NKC, _B, _L), lambda t: (t, 0, 0, 0)),
            pl.BlockSpec((_NKR, _L, 1), lambda t: (0, 0, 0)),
            pl.BlockSpec((_NKR, _L, 1), lambda t: (0, 0, 0)),
            pl.BlockSpec((_NO, _L, 1), lambda t: (0, 0, 0)),
            pl.BlockSpec((_NO, _L, 1), lambda t: (0, 0, 0)),
        ],
        out_specs=[
            pl.BlockSpec((1, 6, _L, _B), lambda t: (t, 0, 0, 0)),
            pl.BlockSpec((1, _NO, _L, _B), lambda t: (t, 0, 0, 0)),
        ],
        out_shape=[
            jax.ShapeDtypeStruct((_T, 6, _L, _B), jnp.float32),
            jax.ShapeDtypeStruct((_T, _NO, _L, _B), jnp.float32),
        ],
        compiler_params=pltpu.CompilerParams(
            dimension_semantics=("parallel",)),
    )(krT, obsT, cat4, wkT, bkT, woT, boT)


def kernel(static, known_real, known_categorical, observed, static_tables,
           known_tables, known_dense_w, known_dense_b, observed_dense_w,
           observed_dense_b):
    sflat = static_tables.reshape(_NST * _V, _L)
    kflat = known_tables.reshape(_NKC * _V, _L)

    krT = jnp.transpose(known_real, (1, 2, 0))
    obsT = jnp.transpose(observed, (1, 2, 0))
    kcT = jnp.transpose(known_categorical, (1, 2, 0))
    staticT = jnp.transpose(static, (1, 2, 0))

    offs_s = (jnp.arange(_NST, dtype=jnp.int32) * _V)[:, None]
    sidx = (staticT[0].astype(jnp.int32) + offs_s).reshape(-1)
    offs_k = (jnp.arange(_NKC, dtype=jnp.int32) * _V)[None, :, None]
    kidx = (kcT.astype(jnp.int32) + offs_k).reshape(-1)

    static_rows, cat_rows = _sc_gather(sflat, kflat, sidx, kidx)

    wkT = jnp.transpose(known_dense_w, (0, 2, 1))
    bkT = known_dense_b[..., None]
    woT = jnp.transpose(observed_dense_w, (0, 2, 1))
    boT = observed_dense_b[..., None]

    known_t, obs_t = _tc_assemble(
        krT, obsT, cat_rows.reshape(_T, _NKC, _B, _L), wkT, bkT, woT, boT)

    return (static_rows.reshape(_NST, _B, _L).transpose(1, 0, 2),
            known_t.transpose(3, 0, 2, 1),
            obs_t.transpose(3, 0, 2, 1))


# speedup vs baseline: 3.1008x; 1.1004x over previous
import functools

import jax
import jax.numpy as jnp
from jax import lax
from jax.experimental import pallas as pl
from jax.experimental.pallas import tpu as pltpu
from jax.experimental.pallas import tpu_sc as plsc

_B, _T, _L, _V = 1024, 200, 32, 100000
_NST, _NKC, _NKR, _NO = 4, 2, 4, 3
_BT = _B * _T
_NC, _NSUB = 2, 16
_NW = _NC * _NSUB
_CH = 128
_SROWS = (_B * _NST) // _NW
_KROWS = (_NKC * _BT) // _NW
_KCHUNKS = _KROWS // _CH
_NBUF = 4


def _sc_gather(sflat, kflat, sidx, kidx):
    mesh = plsc.VectorSubcoreMesh(
        core_axis_name="c", subcore_axis_name="s",
        num_cores=_NC, num_subcores=_NSUB)

    @functools.partial(
        pl.kernel,
        out_type=(
            jax.ShapeDtypeStruct((_B * _NST, _L), jnp.float32),
            jax.ShapeDtypeStruct((_NKC * _BT, _L), jnp.float32),
        ),
        mesh=mesh,
        scratch_types=(
            pltpu.VMEM((_CH,), jnp.int32),
            pltpu.VMEM((_NBUF, _CH), jnp.int32),
            pltpu.VMEM((_CH, _L), jnp.float32),
            pltpu.VMEM((_NBUF, _CH, _L), jnp.float32),
            pltpu.SemaphoreType.DMA,
            pltpu.SemaphoreType.DMA,
            pltpu.SemaphoreType.DMA,
            pltpu.SemaphoreType.DMA,
            pltpu.SemaphoreType.DMA,
        ),
        compiler_params=pltpu.CompilerParams(use_tc_tiling_on_sc=False),
    )
    def gk(sflat_h, kflat_h, sidx_h, kidx_h, sout_h, cout_h,
           sidx_v, kidx_v, srows_v, rows_v, ssem, sem0, sem1, sem2, sem3):
        sems = (sem0, sem1, sem2, sem3)
        w = lax.axis_index("s") * _NC + lax.axis_index("c")
        sbase = w * _SROWS
        kbase = w * _KROWS

        def start(j, b):
            pltpu.sync_copy(kidx_h.at[pl.ds(kbase + j * _CH, _CH)],
                            kidx_v.at[b])
            return pltpu.async_copy(kflat_h.at[kidx_v.at[b]], rows_v.at[b],
                                    sems[b])

        pltpu.sync_copy(sidx_h.at[pl.ds(sbase, _CH)], sidx_v)
        scp = pltpu.async_copy(sflat_h.at[sidx_v], srows_v, ssem)
        for b in range(_NBUF - 1):
            start(b, b)
        scp.wait()
        pltpu.sync_copy(srows_v, sout_h.at[pl.ds(sbase, _CH)])

        def body(p, carry):
            for b in range(_NBUF):
                j = p * _NBUF + b
                nxt = j + (_NBUF - 1)
                bn = (b + _NBUF - 1) % _NBUF

                @pl.when(nxt < _KCHUNKS)
                def _():
                    start(nxt, bn)

                pltpu.make_async_copy(kflat_h.at[kidx_v.at[b]],
                                      rows_v.at[b], sems[b]).wait()
                pltpu.sync_copy(rows_v.at[b],
                                cout_h.at[pl.ds(kbase + j * _CH, _CH)])
            return carry

        lax.fori_loop(0, _KCHUNKS // _NBUF, body, 0)

    return gk(sflat, kflat, sidx, kidx)


def _tc_assemble(krT, obsT, cat4, wkT, bkT, woT, boT):
    def body(kr_ref, obs_ref, cat_ref, wk_ref, bk_ref, wo_ref, bo_ref,
             kout_ref, oout_ref):
        for j in range(_NKR):
            kout_ref[0, j] = wk_ref[j] * kr_ref[0, j][None, :] + bk_ref[j]
        for f in range(_NKC):
            kout_ref[0, _NKR + f] = jnp.transpose(cat_ref[0, f], (1, 0))
        for i in range(_NO):
            oout_ref[0, i] = wo_ref[i] * obs_ref[0, i][None, :] + bo_ref[i]

    return pl.pallas_call(
        body,
        grid=(_T,),
        in_specs=[
            pl.BlockSpec((1, _NKR, _B), lambda t: (t, 0, 0)),
            pl.BlockSpec((1, _NO, _B), lambda t: (t, 0, 0)),
            pl.BlockSpec((1, _NKC, _B, _L), lambda t: (t, 0, 0, 0)),
            pl.BlockSpec((_NKR, _L, 1), lambda t: (0, 0, 0)),
            pl.BlockSpec((_NKR, _L, 1), lambda t: (0, 0, 0)),
            pl.BlockSpec((_NO, _L, 1), lambda t: (0, 0, 0)),
            pl.BlockSpec((_NO, _L, 1), lambda t: (0, 0, 0)),
        ],
        out_specs=[
            pl.BlockSpec((1, 6, _L, _B), lambda t: (t, 0, 0, 0)),
            pl.BlockSpec((1, _NO, _L, _B), lambda t: (t, 0, 0, 0)),
        ],
        out_shape=[
            jax.ShapeDtypeStruct((_T, 6, _L, _B), jnp.float32),
            jax.ShapeDtypeStruct((_T, _NO, _L, _B), jnp.float32),
        ],
        compiler_params=pltpu.CompilerParams(
            dimension_semantics=("parallel",)),
    )(krT, obsT, cat4, wkT, bkT, woT, boT)


def kernel(static, known_real, known_categorical, observed, static_tables,
           known_tables, known_dense_w, known_dense_b, observed_dense_w,
           observed_dense_b):
    sflat = static_tables.reshape(_NST * _V, _L)
    kflat = known_tables.reshape(_NKC * _V, _L)

    krT = jnp.transpose(known_real, (1, 2, 0))
    obsT = jnp.transpose(observed, (1, 2, 0))
    kcT = jnp.transpose(known_categorical, (1, 2, 0))
    staticT = jnp.transpose(static, (1, 2, 0))

    offs_s = (jnp.arange(_NST, dtype=jnp.int32) * _V)[:, None]
    sidx = (staticT[0].astype(jnp.int32) + offs_s).reshape(-1)
    offs_k = (jnp.arange(_NKC, dtype=jnp.int32) * _V)[None, :, None]
    kidx = (kcT.astype(jnp.int32) + offs_k).reshape(-1)

    static_rows, cat_rows = _sc_gather(sflat, kflat, sidx, kidx)

    wkT = jnp.transpose(known_dense_w, (0, 2, 1))
    bkT = known_dense_b[..., None]
    woT = jnp.transpose(observed_dense_w, (0, 2, 1))
    boT = observed_dense_b[..., None]

    known_t, obs_t = _tc_assemble(
        krT, obsT, cat_rows.reshape(_T, _NKC, _B, _L), wkT, bkT, woT, boT)

    return (static_rows.reshape(_NST, _B, _L).transpose(1, 0, 2),
            known_t.transpose(3, 0, 2, 1),
            obs_t.transpose(3, 0, 2, 1))
